# baseline (speedup 1.0000x reference)
import functools

import jax
import jax.numpy as jnp
from jax import lax
from jax.experimental import pallas as pl
from jax.experimental.pallas import tpu as pltpu
from jax.experimental.pallas import tpu_sc as plsc

NC = 2
NS = 16
NW = NC * NS


def _gather_rows_sc(x, idx, chunk):
    e, w = idx.shape[0], x.shape[1]
    per_w = e // NW
    n_ch = per_w // chunk
    mesh = plsc.VectorSubcoreMesh(core_axis_name="c", subcore_axis_name="s")

    @functools.partial(
        pl.kernel,
        out_type=jax.ShapeDtypeStruct((e, w), jnp.float32),
        mesh=mesh,
        scratch_types=[
            pltpu.VMEM((chunk,), jnp.int32),
            pltpu.VMEM((chunk, w), jnp.float32),
            pltpu.SemaphoreType.DMA,
        ],
        compiler_params=pltpu.CompilerParams(use_tc_tiling_on_sc=False),
    )
    def k(x_hbm, idx_hbm, out_hbm, idx_v, rows_v, sem):
        wid = lax.axis_index("s") * NC + lax.axis_index("c")
        base = wid * per_w

        def body(j, carry):
            off = base + j * chunk
            pltpu.sync_copy(idx_hbm.at[pl.ds(off, chunk)], idx_v)
            pltpu.async_copy(x_hbm.at[idx_v], rows_v, sem).wait()
            pltpu.sync_copy(rows_v, out_hbm.at[pl.ds(off, chunk)])
            return carry

        lax.fori_loop(0, n_ch, body, 0)

    return k(x, idx)


def _transpose_pack_sc(eaT, chunk):
    d, e = eaT.shape
    per_w = e // NW
    n_ch = per_w // chunk
    mesh = plsc.VectorSubcoreMesh(core_axis_name="c", subcore_axis_name="s")

    @functools.partial(
        pl.kernel,
        out_type=jax.ShapeDtypeStruct((e // 8, 8 * d), jnp.float32),
        mesh=mesh,
        scratch_types=[
            pltpu.VMEM((d, chunk), jnp.float32),
            pltpu.VMEM((chunk // 8, 8 * d), jnp.float32),
        ],
        compiler_params=pltpu.CompilerParams(use_tc_tiling_on_sc=False,
                                             needs_layout_passes=False),
    )
    def k(eaT_hbm, out_hbm, slab_v, rows_v):
        wid = lax.axis_index("s") * NC + lax.axis_index("c")
        base = wid * per_w
        lanes = lax.iota(jnp.int32, 16)
        lanehi = lax.shift_right_logical(lanes, 3)
        lanelow = lax.shift_left(lax.bitwise_and(lanes, 7), 4)
        cols = [lanelow + jnp.int32(k_) for k_ in range(d)]

        def body(j, carry):
            off = base + j * chunk
            pltpu.sync_copy(eaT_hbm.at[:, pl.ds(off, chunk)], slab_v)

            def grp(g, c2):
                row0 = lanehi + g * 2
                for k_ in range(d):
                    v = slab_v[k_, pl.ds(g * 16, 16)]
                    plsc.store_scatter(rows_v, [row0, cols[k_]], v)
                return c2

            lax.fori_loop(0, chunk // 16, grp, 0)
            pltpu.sync_copy(rows_v, out_hbm.at[pl.ds(off // 8, chunk // 8)])
            return carry

        lax.fori_loop(0, n_ch, body, 0)

    return k(eaT)


def _scatter_add_sc(msgs, idx, n_nodes, chunk):
    e, w = msgs.shape
    per_w = e // NW
    n_ch = per_w // chunk
    rows_per_tile = n_nodes // NS
    mesh = plsc.VectorSubcoreMesh(core_axis_name="c", subcore_axis_name="s")
    zeros = jnp.zeros((n_nodes, w), jnp.float32)

    @functools.partial(
        pl.kernel,
        out_type=jax.ShapeDtypeStruct((NC, n_nodes, w), jnp.float32),
        mesh=mesh,
        scratch_types=[
            pltpu.VMEM((chunk,), jnp.int32),
            pltpu.VMEM((chunk, w), jnp.float32),
            pltpu.VMEM_SHARED((n_nodes, w), jnp.float32),
            pltpu.SemaphoreType.DMA,
        ],
        compiler_params=pltpu.CompilerParams(use_tc_tiling_on_sc=False),
    )
    def k(msg_hbm, idx_hbm, zeros_hbm, out_hbm, idx_v, rows_v, acc_sh, sem):
        c = lax.axis_index("c")
        s = lax.axis_index("s")
        r0 = s * rows_per_tile
        pltpu.sync_copy(zeros_hbm.at[pl.ds(r0, rows_per_tile)],
                        acc_sh.at[pl.ds(r0, rows_per_tile)])
        plsc.subcore_barrier()

        base = (c * NS + s) * per_w

        def body(j, carry):
            off = base + j * chunk
            pltpu.sync_copy(idx_hbm.at[pl.ds(off, chunk)], idx_v)
            pltpu.sync_copy(msg_hbm.at[pl.ds(off, chunk)], rows_v)
            pltpu.sync_copy(rows_v, acc_sh.at[idx_v], add=True)
            return carry

        lax.fori_loop(0, n_ch, body, 0)
        plsc.subcore_barrier()
        pltpu.sync_copy(acc_sh.at[pl.ds(r0, rows_per_tile)],
                        out_hbm.at[c, pl.ds(r0, rows_per_tile)])

    return k(msgs, idx, zeros)


def _messages_tc(ea8, xj8, Rp, Tp, W2p, B0p, blk8):
    e8, pw = ea8.shape

    def body(ea_ref, xj_ref, r_ref, t_ref, w2_ref, b0_ref, out_ref):
        ea_b = ea_ref[...]
        xj_b = xj_ref[...]
        z = (jnp.dot(ea_b, r_ref[...], preferred_element_type=jnp.float32)
             * jnp.dot(xj_b, t_ref[...], preferred_element_type=jnp.float32))
        out_ref[...] = (
            jnp.dot(z, w2_ref[...], preferred_element_type=jnp.float32)
            + jnp.dot(xj_b, b0_ref[...], preferred_element_type=jnp.float32))

    zw = Rp.shape[1]
    return pl.pallas_call(
        body,
        grid=(e8 // blk8,),
        in_specs=[
            pl.BlockSpec((blk8, pw), lambda i: (i, 0)),
            pl.BlockSpec((blk8, pw), lambda i: (i, 0)),
            pl.BlockSpec((pw, zw), lambda i: (0, 0)),
            pl.BlockSpec((pw, zw), lambda i: (0, 0)),
            pl.BlockSpec((zw, pw), lambda i: (0, 0)),
            pl.BlockSpec((pw, pw), lambda i: (0, 0)),
        ],
        out_specs=pl.BlockSpec((blk8, pw), lambda i: (i, 0)),
        out_shape=jax.ShapeDtypeStruct((e8, pw), jnp.float32),
    )(ea8, xj8, Rp, Tp, W2p, B0p)


def kernel(x, senders, receivers, edge_attr, W_nn, b_nn):
    n_nodes, w = x.shape
    d_dim = edge_attr.shape[1]
    senders = senders.astype(jnp.int32)
    receivers = receivers.astype(jnp.int32)
    eye = jnp.eye(w, dtype=jnp.float32)
    i8 = jnp.eye(8, dtype=jnp.float32)
    R = jnp.repeat(eye, w, axis=1)
    T = jnp.tile(eye, (1, d_dim))
    W2 = W_nn.reshape(d_dim * w, w)
    b0 = b_nn.reshape(w, w)
    Rp = jnp.kron(i8, R)
    Tp = jnp.kron(i8, T)
    W2p = jnp.kron(i8, W2)
    B0p = jnp.kron(i8, b0)

    e = senders.shape[0]
    ea8 = _transpose_pack_sc(edge_attr.T, chunk=1000)
    xj = _gather_rows_sc(x, senders, chunk=5000)
    xj8 = xj.reshape(e // 8, 8 * w)
    msgs8 = _messages_tc(ea8, xj8, Rp, Tp, W2p, B0p, blk8=800)
    msgs = msgs8.reshape(e, w)
    partials = _scatter_add_sc(msgs, receivers, n_nodes, chunk=1000)
    p8 = partials.reshape(NC, n_nodes // 8, 8 * w)
    return (p8[0] + p8[1]).reshape(n_nodes, w)


# speedup vs baseline: 1.8736x; 1.8736x over previous
import functools

import jax
import jax.numpy as jnp
from jax import lax
from jax.experimental import pallas as pl
from jax.experimental.pallas import tpu as pltpu
from jax.experimental.pallas import tpu_sc as plsc

NC = 2
NS = 16
NW = NC * NS


def _gather_rows_sc(x, idx, chunk):
    e, w = idx.shape[0], x.shape[1]
    per_w = e // NW
    n_ch = per_w // chunk
    mesh = plsc.VectorSubcoreMesh(core_axis_name="c", subcore_axis_name="s")

    @functools.partial(
        pl.kernel,
        out_type=jax.ShapeDtypeStruct((e, w), jnp.float32),
        mesh=mesh,
        scratch_types=[
            pltpu.VMEM((chunk,), jnp.int32),
            pltpu.VMEM((chunk, w), jnp.float32),
            pltpu.SemaphoreType.DMA,
        ],
        compiler_params=pltpu.CompilerParams(use_tc_tiling_on_sc=False),
    )
    def k(x_hbm, idx_hbm, out_hbm, idx_v, rows_v, sem):
        wid = lax.axis_index("s") * NC + lax.axis_index("c")
        base = wid * per_w

        def body(j, carry):
            off = base + j * chunk
            pltpu.sync_copy(idx_hbm.at[pl.ds(off, chunk)], idx_v)
            pltpu.async_copy(x_hbm.at[idx_v], rows_v, sem).wait()
            pltpu.sync_copy(rows_v, out_hbm.at[pl.ds(off, chunk)])
            return carry

        lax.fori_loop(0, n_ch, body, 0)

    return k(x, idx)


def _transpose_pack_sc(eaT, chunk):
    d, e = eaT.shape
    per_w = e // NW
    n_ch = per_w // chunk
    mesh = plsc.VectorSubcoreMesh(core_axis_name="c", subcore_axis_name="s")

    @functools.partial(
        pl.kernel,
        out_type=jax.ShapeDtypeStruct((e // 8, 8 * d), jnp.float32),
        mesh=mesh,
        scratch_types=[
            pltpu.VMEM((d, chunk), jnp.float32),
            pltpu.VMEM((chunk // 8, 8 * d), jnp.float32),
        ],
        compiler_params=pltpu.CompilerParams(use_tc_tiling_on_sc=False,
                                             needs_layout_passes=False),
    )
    def k(eaT_hbm, out_hbm, slab_v, rows_v):
        wid = lax.axis_index("s") * NC + lax.axis_index("c")
        base = wid * per_w
        lanes = lax.iota(jnp.int32, 16)
        lanehi = lax.shift_right_logical(lanes, 3)
        lanelow = lax.shift_left(lax.bitwise_and(lanes, 7), 4)
        cols = [lanelow + jnp.int32(k_) for k_ in range(d)]

        def body(j, carry):
            off = base + j * chunk
            pltpu.sync_copy(eaT_hbm.at[:, pl.ds(off, chunk)], slab_v)

            def grp(g, c2):
                row0 = lanehi + g * 2
                for k_ in range(d):
                    v = slab_v[k_, pl.ds(g * 16, 16)]
                    plsc.store_scatter(rows_v, [row0, cols[k_]], v)
                return c2

            lax.fori_loop(0, chunk // 16, grp, 0)
            pltpu.sync_copy(rows_v, out_hbm.at[pl.ds(off // 8, chunk // 8)])
            return carry

        lax.fori_loop(0, n_ch, body, 0)

    return k(eaT)


def _scatter_add_sc(msgs, idx, n_nodes, chunk):
    e, w = msgs.shape
    per_w = e // NW
    n_ch = per_w // chunk
    rows_per_tile = n_nodes // NS
    mesh = plsc.VectorSubcoreMesh(core_axis_name="c", subcore_axis_name="s")
    zeros = jnp.zeros((n_nodes, w), jnp.float32)

    @functools.partial(
        pl.kernel,
        out_type=jax.ShapeDtypeStruct((NC, n_nodes, w), jnp.float32),
        mesh=mesh,
        scratch_types=[
            pltpu.VMEM((chunk,), jnp.int32),
            pltpu.VMEM((chunk, w), jnp.float32),
            pltpu.VMEM_SHARED((n_nodes, w), jnp.float32),
            pltpu.SemaphoreType.DMA,
        ],
        compiler_params=pltpu.CompilerParams(use_tc_tiling_on_sc=False),
    )
    def k(msg_hbm, idx_hbm, zeros_hbm, out_hbm, idx_v, rows_v, acc_sh, sem):
        c = lax.axis_index("c")
        s = lax.axis_index("s")
        r0 = s * rows_per_tile
        pltpu.sync_copy(zeros_hbm.at[pl.ds(r0, rows_per_tile)],
                        acc_sh.at[pl.ds(r0, rows_per_tile)])
        plsc.subcore_barrier()

        base = (c * NS + s) * per_w

        def body(j, carry):
            off = base + j * chunk
            pltpu.sync_copy(idx_hbm.at[pl.ds(off, chunk)], idx_v)
            pltpu.sync_copy(msg_hbm.at[pl.ds(off, chunk)], rows_v)
            pltpu.sync_copy(rows_v, acc_sh.at[idx_v], add=True)
            return carry

        lax.fori_loop(0, n_ch, body, 0)
        plsc.subcore_barrier()
        pltpu.sync_copy(acc_sh.at[pl.ds(r0, rows_per_tile)],
                        out_hbm.at[c, pl.ds(r0, rows_per_tile)])

    return k(msgs, idx, zeros)


def _messages_tc(ea8, xj8, Rp, Tp, W2p, B0p, blk8):
    e8, pw = ea8.shape

    def body(ea_ref, xj_ref, r_ref, t_ref, w2_ref, b0_ref, out_ref):
        ea_b = ea_ref[...]
        xj_b = xj_ref[...]
        acc = jnp.dot(xj_b, b0_ref[...], preferred_element_type=jnp.float32)
        for p in range(8):
            zp = (jnp.dot(ea_b, r_ref[pl.ds(0, 128), pl.ds(p * 256, 256)],
                          preferred_element_type=jnp.float32)
                  * jnp.dot(xj_b, t_ref[pl.ds(0, 128), pl.ds(p * 256, 256)],
                            preferred_element_type=jnp.float32))
            acc = acc + jnp.dot(zp, w2_ref[pl.ds(p * 256, 256), pl.ds(0, 128)],
                                preferred_element_type=jnp.float32)
        out_ref[...] = acc

    zw = Rp.shape[1]
    return pl.pallas_call(
        body,
        grid=(e8 // blk8,),
        in_specs=[
            pl.BlockSpec((blk8, pw), lambda i: (i, 0)),
            pl.BlockSpec((blk8, pw), lambda i: (i, 0)),
            pl.BlockSpec((pw, zw), lambda i: (0, 0)),
            pl.BlockSpec((pw, zw), lambda i: (0, 0)),
            pl.BlockSpec((zw, pw), lambda i: (0, 0)),
            pl.BlockSpec((pw, pw), lambda i: (0, 0)),
        ],
        out_specs=pl.BlockSpec((blk8, pw), lambda i: (i, 0)),
        out_shape=jax.ShapeDtypeStruct((e8, pw), jnp.float32),
    )(ea8, xj8, Rp, Tp, W2p, B0p)


def kernel(x, senders, receivers, edge_attr, W_nn, b_nn):
    n_nodes, w = x.shape
    d_dim = edge_attr.shape[1]
    senders = senders.astype(jnp.int32)
    receivers = receivers.astype(jnp.int32)
    eye = jnp.eye(w, dtype=jnp.float32)
    i8 = jnp.eye(8, dtype=jnp.float32)
    R = jnp.repeat(eye, w, axis=1)
    T = jnp.tile(eye, (1, d_dim))
    W2 = W_nn.reshape(d_dim * w, w)
    b0 = b_nn.reshape(w, w)
    Rp = jnp.kron(i8, R)
    Tp = jnp.kron(i8, T)
    W2p = jnp.kron(i8, W2)
    B0p = jnp.kron(i8, b0)

    e = senders.shape[0]
    ea8 = edge_attr.reshape(e // 8, 8 * d_dim)
    xj = _gather_rows_sc(x, senders, chunk=5000)
    xj8 = xj.reshape(e // 8, 8 * w)
    msgs8 = _messages_tc(ea8, xj8, Rp, Tp, W2p, B0p, blk8=800)
    msgs = msgs8.reshape(e, w)
    partials = _scatter_add_sc(msgs, receivers, n_nodes, chunk=1000)
    p8 = partials.reshape(NC, n_nodes // 8, 8 * w)
    return (p8[0] + p8[1]).reshape(n_nodes, w)


# speedup vs baseline: 1.9426x; 1.0368x over previous
import functools

import jax
import jax.numpy as jnp
from jax import lax
from jax.experimental import pallas as pl
from jax.experimental.pallas import tpu as pltpu
from jax.experimental.pallas import tpu_sc as plsc

NC = 2
NS = 16
NW = NC * NS


def _gather_rows_sc(x, idx, chunk):
    e, w = idx.shape[0], x.shape[1]
    per_w = e // NW
    n_ch = per_w // chunk
    mesh = plsc.VectorSubcoreMesh(core_axis_name="c", subcore_axis_name="s")

    @functools.partial(
        pl.kernel,
        out_type=jax.ShapeDtypeStruct((e, w), jnp.float32),
        mesh=mesh,
        scratch_types=[
            pltpu.VMEM((chunk,), jnp.int32),
            pltpu.VMEM((chunk, w), jnp.float32),
            pltpu.SemaphoreType.DMA,
        ],
        compiler_params=pltpu.CompilerParams(use_tc_tiling_on_sc=False),
    )
    def k(x_hbm, idx_hbm, out_hbm, idx_v, rows_v, sem):
        wid = lax.axis_index("s") * NC + lax.axis_index("c")
        base = wid * per_w

        def body(j, carry):
            off = base + j * chunk
            pltpu.sync_copy(idx_hbm.at[pl.ds(off, chunk)], idx_v)
            pltpu.async_copy(x_hbm.at[idx_v], rows_v, sem).wait()
            pltpu.sync_copy(rows_v, out_hbm.at[pl.ds(off, chunk)])
            return carry

        lax.fori_loop(0, n_ch, body, 0)

    return k(x, idx)


def _compact_pack_sc(ea, chunk):
    e, w = ea.shape
    per_w = e // NW
    n_ch = per_w // chunk
    mesh = plsc.VectorSubcoreMesh(core_axis_name="c", subcore_axis_name="s")

    @functools.partial(
        pl.kernel,
        out_type=jax.ShapeDtypeStruct((e // 8, 8 * w), jnp.float32),
        mesh=mesh,
        scratch_types=[
            pltpu.VMEM((chunk, w), jnp.float32),
        ],
        compiler_params=pltpu.CompilerParams(use_tc_tiling_on_sc=True),
    )
    def k(ea_hbm, out_hbm, rows_v):
        wid = lax.axis_index("s") * NC + lax.axis_index("c")
        base = wid * per_w

        def body(j, carry):
            off = base + j * chunk
            pltpu.sync_copy(ea_hbm.at[pl.ds(off, chunk)], rows_v)
            pltpu.sync_copy(rows_v, out_hbm.at[pl.ds(off // 8, chunk // 8)])
            return carry

        lax.fori_loop(0, n_ch, body, 0)

    return k(ea)


def _scatter_add_sc(msgs, idx, n_nodes, chunk):
    e, w = msgs.shape
    per_w = e // NW
    n_ch = per_w // chunk
    rows_per_tile = n_nodes // NS
    mesh = plsc.VectorSubcoreMesh(core_axis_name="c", subcore_axis_name="s")
    zeros = jnp.zeros((n_nodes, w), jnp.float32)

    @functools.partial(
        pl.kernel,
        out_type=jax.ShapeDtypeStruct((NC, n_nodes, w), jnp.float32),
        mesh=mesh,
        scratch_types=[
            pltpu.VMEM((chunk,), jnp.int32),
            pltpu.VMEM((chunk, w), jnp.float32),
            pltpu.VMEM_SHARED((n_nodes, w), jnp.float32),
            pltpu.SemaphoreType.DMA,
        ],
        compiler_params=pltpu.CompilerParams(use_tc_tiling_on_sc=False),
    )
    def k(msg_hbm, idx_hbm, zeros_hbm, out_hbm, idx_v, rows_v, acc_sh, sem):
        c = lax.axis_index("c")
        s = lax.axis_index("s")
        r0 = s * rows_per_tile
        pltpu.sync_copy(zeros_hbm.at[pl.ds(r0, rows_per_tile)],
                        acc_sh.at[pl.ds(r0, rows_per_tile)])
        plsc.subcore_barrier()

        base = (c * NS + s) * per_w

        def body(j, carry):
            off = base + j * chunk
            pltpu.sync_copy(idx_hbm.at[pl.ds(off, chunk)], idx_v)
            pltpu.sync_copy(msg_hbm.at[pl.ds(off, chunk)], rows_v)
            pltpu.sync_copy(rows_v, acc_sh.at[idx_v], add=True)
            return carry

        lax.fori_loop(0, n_ch, body, 0)
        plsc.subcore_barrier()
        pltpu.sync_copy(acc_sh.at[pl.ds(r0, rows_per_tile)],
                        out_hbm.at[c, pl.ds(r0, rows_per_tile)])

    return k(msgs, idx, zeros)


def _messages_tc(ea8, xj8, Rp, Tp, W2p, B0p, blk8):
    e8, pw = ea8.shape

    def body(ea_ref, xj_ref, r_ref, t_ref, w2_ref, b0_ref, out_ref):
        ea_b = ea_ref[...]
        xj_b = xj_ref[...]
        acc = jnp.dot(xj_b, b0_ref[...], preferred_element_type=jnp.float32)
        for p in range(8):
            zp = (jnp.dot(ea_b, r_ref[pl.ds(0, 128), pl.ds(p * 256, 256)],
                          preferred_element_type=jnp.float32)
                  * jnp.dot(xj_b, t_ref[pl.ds(0, 128), pl.ds(p * 256, 256)],
                            preferred_element_type=jnp.float32))
            acc = acc + jnp.dot(zp, w2_ref[pl.ds(p * 256, 256), pl.ds(0, 128)],
                                preferred_element_type=jnp.float32)
        out_ref[...] = acc

    zw = Rp.shape[1]
    return pl.pallas_call(
        body,
        grid=(e8 // blk8,),
        in_specs=[
            pl.BlockSpec((blk8, pw), lambda i: (i, 0)),
            pl.BlockSpec((blk8, pw), lambda i: (i, 0)),
            pl.BlockSpec((pw, zw), lambda i: (0, 0)),
            pl.BlockSpec((pw, zw), lambda i: (0, 0)),
            pl.BlockSpec((zw, pw), lambda i: (0, 0)),
            pl.BlockSpec((pw, pw), lambda i: (0, 0)),
        ],
        out_specs=pl.BlockSpec((blk8, pw), lambda i: (i, 0)),
        out_shape=jax.ShapeDtypeStruct((e8, pw), jnp.float32),
    )(ea8, xj8, Rp, Tp, W2p, B0p)


def kernel(x, senders, receivers, edge_attr, W_nn, b_nn):
    n_nodes, w = x.shape
    d_dim = edge_attr.shape[1]
    senders = senders.astype(jnp.int32)
    receivers = receivers.astype(jnp.int32)
    eye = jnp.eye(w, dtype=jnp.float32)
    i8 = jnp.eye(8, dtype=jnp.float32)
    R = jnp.repeat(eye, w, axis=1)
    T = jnp.tile(eye, (1, d_dim))
    W2 = W_nn.reshape(d_dim * w, w)
    b0 = b_nn.reshape(w, w)
    Rp = jnp.kron(i8, R)
    Tp = jnp.kron(i8, T)
    W2p = jnp.kron(i8, W2)
    B0p = jnp.kron(i8, b0)

    e = senders.shape[0]
    ea8 = edge_attr.reshape(e // 8, 8 * d_dim)
    xj = _gather_rows_sc(x, senders, chunk=5000)
    xj8 = xj.reshape(e // 8, 8 * w)
    msgs8 = _messages_tc(ea8, xj8, Rp, Tp, W2p, B0p, blk8=2000)
    msgs = msgs8.reshape(e, w)
    partials = _scatter_add_sc(msgs, receivers, n_nodes, chunk=1000)
    p8 = partials.reshape(NC, n_nodes // 8, 8 * w)
    return (p8[0] + p8[1]).reshape(n_nodes, w)


# speedup vs baseline: 2.0660x; 1.0635x over previous
import functools

import jax
import jax.numpy as jnp
from jax import lax
from jax.experimental import pallas as pl
from jax.experimental.pallas import tpu as pltpu
from jax.experimental.pallas import tpu_sc as plsc

NC = 2
NS = 16
NW = NC * NS


def _gather_rows_sc(x, idx, chunk):
    e, w = idx.shape[0], x.shape[1]
    per_w = e // NW
    n_ch = per_w // chunk
    mesh = plsc.VectorSubcoreMesh(core_axis_name="c", subcore_axis_name="s")

    @functools.partial(
        pl.kernel,
        out_type=jax.ShapeDtypeStruct((e, w), jnp.float32),
        mesh=mesh,
        scratch_types=[
            pltpu.VMEM((chunk,), jnp.int32),
            pltpu.VMEM((chunk, w), jnp.float32),
            pltpu.SemaphoreType.DMA,
        ],
        compiler_params=pltpu.CompilerParams(use_tc_tiling_on_sc=False),
    )
    def k(x_hbm, idx_hbm, out_hbm, idx_v, rows_v, sem):
        wid = lax.axis_index("s") * NC + lax.axis_index("c")
        base = wid * per_w

        def body(j, carry):
            off = base + j * chunk
            pltpu.sync_copy(idx_hbm.at[pl.ds(off, chunk)], idx_v)
            pltpu.async_copy(x_hbm.at[idx_v], rows_v, sem).wait()
            pltpu.sync_copy(rows_v, out_hbm.at[pl.ds(off, chunk)])
            return carry

        lax.fori_loop(0, n_ch, body, 0)

    return k(x, idx)


def _compact_pack_sc(ea, chunk):
    e, w = ea.shape
    per_w = e // NW
    n_ch = per_w // chunk
    mesh = plsc.VectorSubcoreMesh(core_axis_name="c", subcore_axis_name="s")

    @functools.partial(
        pl.kernel,
        out_type=jax.ShapeDtypeStruct((e // 8, 8 * w), jnp.float32),
        mesh=mesh,
        scratch_types=[
            pltpu.VMEM((chunk, w), jnp.float32),
        ],
        compiler_params=pltpu.CompilerParams(use_tc_tiling_on_sc=True),
    )
    def k(ea_hbm, out_hbm, rows_v):
        wid = lax.axis_index("s") * NC + lax.axis_index("c")
        base = wid * per_w

        def body(j, carry):
            off = base + j * chunk
            pltpu.sync_copy(ea_hbm.at[pl.ds(off, chunk)], rows_v)
            pltpu.sync_copy(rows_v, out_hbm.at[pl.ds(off // 8, chunk // 8)])
            return carry

        lax.fori_loop(0, n_ch, body, 0)

    return k(ea)


def _scatter_add_sc(msgs, idx, n_nodes, chunk):
    e, w = msgs.shape
    per_w = e // NW
    n_ch = per_w // chunk
    rows_per_tile = n_nodes // NS
    mesh = plsc.VectorSubcoreMesh(core_axis_name="c", subcore_axis_name="s")
    zeros = jnp.zeros((n_nodes, w), jnp.float32)

    @functools.partial(
        pl.kernel,
        out_type=jax.ShapeDtypeStruct((NC, n_nodes, w), jnp.float32),
        mesh=mesh,
        scratch_types=[
            pltpu.VMEM((chunk,), jnp.int32),
            pltpu.VMEM((chunk, w), jnp.float32),
            pltpu.VMEM_SHARED((n_nodes, w), jnp.float32),
            pltpu.SemaphoreType.DMA,
        ],
        compiler_params=pltpu.CompilerParams(use_tc_tiling_on_sc=False),
    )
    def k(msg_hbm, idx_hbm, zeros_hbm, out_hbm, idx_v, rows_v, acc_sh, sem):
        c = lax.axis_index("c")
        s = lax.axis_index("s")
        r0 = s * rows_per_tile
        pltpu.sync_copy(zeros_hbm.at[pl.ds(r0, rows_per_tile)],
                        acc_sh.at[pl.ds(r0, rows_per_tile)])
        plsc.subcore_barrier()

        base = (c * NS + s) * per_w

        def body(j, carry):
            off = base + j * chunk
            pltpu.sync_copy(idx_hbm.at[pl.ds(off, chunk)], idx_v)
            pltpu.sync_copy(msg_hbm.at[pl.ds(off, chunk)], rows_v)
            pltpu.sync_copy(rows_v, acc_sh.at[idx_v], add=True)
            return carry

        lax.fori_loop(0, n_ch, body, 0)
        plsc.subcore_barrier()
        pltpu.sync_copy(acc_sh.at[pl.ds(r0, rows_per_tile)],
                        out_hbm.at[c, pl.ds(r0, rows_per_tile)])

    return k(msgs, idx, zeros)


def _messages_tc(ea8, xj8, Rp, Tp, W2p, B0p, blk8):
    e8, pw = ea8.shape

    def body(ea_ref, xj_ref, r_ref, t_ref, w2_ref, b0_ref, out_ref):
        ea_b = ea_ref[...]
        xj_b = xj_ref[...]
        acc = jnp.dot(xj_b, b0_ref[...], preferred_element_type=jnp.float32)
        for p in range(8):
            zp = (jnp.dot(ea_b, r_ref[pl.ds(0, 128), pl.ds(p * 256, 256)],
                          preferred_element_type=jnp.float32)
                  * jnp.dot(xj_b, t_ref[pl.ds(0, 128), pl.ds(p * 256, 256)],
                            preferred_element_type=jnp.float32))
            acc = acc + jnp.dot(zp, w2_ref[pl.ds(p * 256, 256), pl.ds(0, 128)],
                                preferred_element_type=jnp.float32)
        out_ref[...] = acc

    zw = Rp.shape[1]
    return pl.pallas_call(
        body,
        grid=(e8 // blk8,),
        in_specs=[
            pl.BlockSpec((blk8, pw), lambda i: (i, 0)),
            pl.BlockSpec((blk8, pw), lambda i: (i, 0)),
            pl.BlockSpec((pw, zw), lambda i: (0, 0)),
            pl.BlockSpec((pw, zw), lambda i: (0, 0)),
            pl.BlockSpec((zw, pw), lambda i: (0, 0)),
            pl.BlockSpec((pw, pw), lambda i: (0, 0)),
        ],
        out_specs=pl.BlockSpec((blk8, pw), lambda i: (i, 0)),
        out_shape=jax.ShapeDtypeStruct((e8, pw), jnp.float32),
    )(ea8, xj8, Rp, Tp, W2p, B0p)


def _sum_partials_tc(p8):
    nc, r, c = p8.shape

    def body(a_ref, b_ref, out_ref):
        out_ref[...] = a_ref[0] + b_ref[0]

    return pl.pallas_call(
        body,
        grid=(1,),
        in_specs=[
            pl.BlockSpec((1, r, c), lambda i: (0, 0, 0)),
            pl.BlockSpec((1, r, c), lambda i: (1, 0, 0)),
        ],
        out_specs=pl.BlockSpec((r, c), lambda i: (0, 0)),
        out_shape=jax.ShapeDtypeStruct((r, c), jnp.float32),
    )(p8, p8)


def kernel(x, senders, receivers, edge_attr, W_nn, b_nn):
    n_nodes, w = x.shape
    d_dim = edge_attr.shape[1]
    senders = senders.astype(jnp.int32)
    receivers = receivers.astype(jnp.int32)
    eye = jnp.eye(w, dtype=jnp.float32)
    i8 = jnp.eye(8, dtype=jnp.float32)
    R = jnp.repeat(eye, w, axis=1)
    T = jnp.tile(eye, (1, d_dim))
    W2 = W_nn.reshape(d_dim * w, w)
    b0 = b_nn.reshape(w, w)
    Rp = jnp.kron(i8, R)
    Tp = jnp.kron(i8, T)
    W2p = jnp.kron(i8, W2)
    B0p = jnp.kron(i8, b0)

    e = senders.shape[0]
    ea8 = edge_attr.reshape(e // 8, 8 * d_dim)
    xj = _gather_rows_sc(x, senders, chunk=5000)
    xj8 = xj.reshape(e // 8, 8 * w)
    msgs8 = _messages_tc(ea8, xj8, Rp, Tp, W2p, B0p, blk8=2000)
    msgs = msgs8.reshape(e, w)
    partials = _scatter_add_sc(msgs, receivers, n_nodes, chunk=1000)
    p8 = partials.reshape(NC, n_nodes // 8, 8 * w)
    return _sum_partials_tc(p8).reshape(n_nodes, w)


# speedup vs baseline: 2.1234x; 1.0278x over previous
import functools

import jax
import jax.numpy as jnp
from jax import lax
from jax.experimental import pallas as pl
from jax.experimental.pallas import tpu as pltpu
from jax.experimental.pallas import tpu_sc as plsc

NC = 2
NS = 16
NW = NC * NS


def _gather_rows_sc(x, idx, chunk):
    e, w = idx.shape[0], x.shape[1]
    per_w = e // NW
    n_ch = per_w // chunk
    mesh = plsc.VectorSubcoreMesh(core_axis_name="c", subcore_axis_name="s")

    @functools.partial(
        pl.kernel,
        out_type=jax.ShapeDtypeStruct((e, w), jnp.float32),
        mesh=mesh,
        scratch_types=[
            pltpu.VMEM((chunk,), jnp.int32),
            pltpu.VMEM((chunk, w), jnp.float32),
            pltpu.SemaphoreType.DMA,
        ],
        compiler_params=pltpu.CompilerParams(use_tc_tiling_on_sc=False),
    )
    def k(x_hbm, idx_hbm, out_hbm, idx_v, rows_v, sem):
        wid = lax.axis_index("s") * NC + lax.axis_index("c")
        base = wid * per_w

        def body(j, carry):
            off = base + j * chunk
            pltpu.sync_copy(idx_hbm.at[pl.ds(off, chunk)], idx_v)
            pltpu.async_copy(x_hbm.at[idx_v], rows_v, sem).wait()
            pltpu.sync_copy(rows_v, out_hbm.at[pl.ds(off, chunk)])
            return carry

        lax.fori_loop(0, n_ch, body, 0)

    return k(x, idx)


def _compact_pack_sc(ea, chunk):
    e, w = ea.shape
    per_w = e // NW
    n_ch = per_w // chunk
    mesh = plsc.VectorSubcoreMesh(core_axis_name="c", subcore_axis_name="s")

    @functools.partial(
        pl.kernel,
        out_type=jax.ShapeDtypeStruct((e // 8, 8 * w), jnp.float32),
        mesh=mesh,
        scratch_types=[
            pltpu.VMEM((chunk, w), jnp.float32),
        ],
        compiler_params=pltpu.CompilerParams(use_tc_tiling_on_sc=True),
    )
    def k(ea_hbm, out_hbm, rows_v):
        wid = lax.axis_index("s") * NC + lax.axis_index("c")
        base = wid * per_w

        def body(j, carry):
            off = base + j * chunk
            pltpu.sync_copy(ea_hbm.at[pl.ds(off, chunk)], rows_v)
            pltpu.sync_copy(rows_v, out_hbm.at[pl.ds(off // 8, chunk // 8)])
            return carry

        lax.fori_loop(0, n_ch, body, 0)

    return k(ea)


def _scatter_add_sc(msgs, idx, n_nodes, chunk, e0=0, e_len=None):
    e_len = msgs.shape[0] if e_len is None else e_len
    w = msgs.shape[1]
    per_w = e_len // NW
    n_ch = per_w // chunk
    rows_per_tile = n_nodes // NS
    mesh = plsc.VectorSubcoreMesh(core_axis_name="c", subcore_axis_name="s")
    zeros = jnp.zeros((n_nodes, w), jnp.float32)

    @functools.partial(
        pl.kernel,
        out_type=jax.ShapeDtypeStruct((NC, n_nodes, w), jnp.float32),
        mesh=mesh,
        scratch_types=[
            pltpu.VMEM((chunk,), jnp.int32),
            pltpu.VMEM((chunk, w), jnp.float32),
            pltpu.VMEM_SHARED((n_nodes, w), jnp.float32),
            pltpu.SemaphoreType.DMA,
        ],
        compiler_params=pltpu.CompilerParams(use_tc_tiling_on_sc=False),
    )
    def k(msg_hbm, idx_hbm, zeros_hbm, out_hbm, idx_v, rows_v, acc_sh, sem):
        c = lax.axis_index("c")
        s = lax.axis_index("s")
        r0 = s * rows_per_tile
        pltpu.sync_copy(zeros_hbm.at[pl.ds(r0, rows_per_tile)],
                        acc_sh.at[pl.ds(r0, rows_per_tile)])
        plsc.subcore_barrier()

        base = (c * NS + s) * per_w

        def body(j, carry):
            off = base + j * chunk
            pltpu.sync_copy(idx_hbm.at[pl.ds(e0 + off, chunk)], idx_v)
            pltpu.sync_copy(msg_hbm.at[pl.ds(off, chunk)], rows_v)
            pltpu.sync_copy(rows_v, acc_sh.at[idx_v], add=True)
            return carry

        lax.fori_loop(0, n_ch, body, 0)
        plsc.subcore_barrier()
        pltpu.sync_copy(acc_sh.at[pl.ds(r0, rows_per_tile)],
                        out_hbm.at[c, pl.ds(r0, rows_per_tile)])

    return k(msgs, idx, zeros)


def _messages_tc(ea8, xj8, Rp, Tp, W2p, B0p, blk8, blk0, n_blocks):
    e8, pw = ea8.shape

    def body(ea_ref, xj_ref, r_ref, t_ref, w2_ref, b0_ref, out_ref):
        ea_b = ea_ref[...]
        xj_b = xj_ref[...]
        acc = jnp.dot(xj_b, b0_ref[...], preferred_element_type=jnp.float32)
        for p in range(8):
            zp = (jnp.dot(ea_b, r_ref[pl.ds(0, 128), pl.ds(p * 256, 256)],
                          preferred_element_type=jnp.float32)
                  * jnp.dot(xj_b, t_ref[pl.ds(0, 128), pl.ds(p * 256, 256)],
                            preferred_element_type=jnp.float32))
            acc = acc + jnp.dot(zp, w2_ref[pl.ds(p * 256, 256), pl.ds(0, 128)],
                                preferred_element_type=jnp.float32)
        out_ref[...] = acc

    zw = Rp.shape[1]
    return pl.pallas_call(
        body,
        grid=(n_blocks,),
        in_specs=[
            pl.BlockSpec((blk8, pw), lambda i: (i + blk0, 0)),
            pl.BlockSpec((blk8, pw), lambda i: (i + blk0, 0)),
            pl.BlockSpec((pw, zw), lambda i: (0, 0)),
            pl.BlockSpec((pw, zw), lambda i: (0, 0)),
            pl.BlockSpec((zw, pw), lambda i: (0, 0)),
            pl.BlockSpec((pw, pw), lambda i: (0, 0)),
        ],
        out_specs=pl.BlockSpec((blk8, pw), lambda i: (i, 0)),
        out_shape=jax.ShapeDtypeStruct((n_blocks * blk8, pw), jnp.float32),
    )(ea8, xj8, Rp, Tp, W2p, B0p)


def _sum_partials_tc(pa, pb):
    nc, r, c = pa.shape

    def body(a0_ref, a1_ref, b0_ref, b1_ref, out_ref):
        out_ref[...] = (a0_ref[0] + a1_ref[0]) + (b0_ref[0] + b1_ref[0])

    return pl.pallas_call(
        body,
        grid=(1,),
        in_specs=[
            pl.BlockSpec((1, r, c), lambda i: (0, 0, 0)),
            pl.BlockSpec((1, r, c), lambda i: (1, 0, 0)),
            pl.BlockSpec((1, r, c), lambda i: (0, 0, 0)),
            pl.BlockSpec((1, r, c), lambda i: (1, 0, 0)),
        ],
        out_specs=pl.BlockSpec((r, c), lambda i: (0, 0)),
        out_shape=jax.ShapeDtypeStruct((r, c), jnp.float32),
    )(pa, pa, pb, pb)


def kernel(x, senders, receivers, edge_attr, W_nn, b_nn):
    n_nodes, w = x.shape
    d_dim = edge_attr.shape[1]
    senders = senders.astype(jnp.int32)
    receivers = receivers.astype(jnp.int32)
    eye = jnp.eye(w, dtype=jnp.float32)
    i8 = jnp.eye(8, dtype=jnp.float32)
    R = jnp.repeat(eye, w, axis=1)
    T = jnp.tile(eye, (1, d_dim))
    W2 = W_nn.reshape(d_dim * w, w)
    b0 = b_nn.reshape(w, w)
    Rp = jnp.kron(i8, R)
    Tp = jnp.kron(i8, T)
    W2p = jnp.kron(i8, W2)
    B0p = jnp.kron(i8, b0)

    e = senders.shape[0]
    ea8 = edge_attr.reshape(e // 8, 8 * d_dim)
    xj = _gather_rows_sc(x, senders, chunk=5000)
    xj8 = xj.reshape(e // 8, 8 * w)
    blk8 = 2000
    nb1 = 24
    e1 = nb1 * blk8 * 8
    nb2 = e // (8 * blk8) - nb1
    m1 = _messages_tc(ea8, xj8, Rp, Tp, W2p, B0p, blk8, 0, nb1)
    m2 = _messages_tc(ea8, xj8, Rp, Tp, W2p, B0p, blk8, nb1, nb2)
    pa = _scatter_add_sc(m1.reshape(e1, w), receivers, n_nodes, chunk=1000,
                         e0=0, e_len=e1)
    pb = _scatter_add_sc(m2.reshape(e - e1, w), receivers, n_nodes, chunk=1000,
                         e0=e1, e_len=e - e1)
    pa8 = pa.reshape(NC, n_nodes // 8, 8 * w)
    pb8 = pb.reshape(NC, n_nodes // 8, 8 * w)
    return _sum_partials_tc(pa8, pb8).reshape(n_nodes, w)
